# baseline (speedup 1.0000x reference)
import functools

import jax
import jax.numpy as jnp
from jax import lax
from jax.experimental import pallas as pl
from jax.experimental.pallas import tpu as pltpu
from jax.experimental.pallas import tpu_sc as plsc

B, D, E = 32768, 4096, 64
TOP_K = 8
BT = 1024
NC = 4
CH = BT // NC

NWORK = 32
TPW = B // NWORK


def _chunk_body(logits, keys_ref, s_ref, allp_ref, c):
    m = jnp.max(logits, axis=1, keepdims=True)
    q = logits - m
    e = jnp.exp(q)
    s = jnp.sum(e, axis=1, keepdims=True)
    allp_ref[pl.ds(c * CH, CH), :] = e / s
    iota = jax.lax.broadcasted_iota(jnp.int32, (CH, E), 1)
    kq = (jnp.maximum(q, -8.0) * (2.0 ** 22)).astype(jnp.int32)
    keys_ref[pl.ds(c * CH, CH), :] = kq * 64 + (63 - iota)
    s_ref[pl.ds(c * CH, CH), :] = s


def _router_block(x_ref, w_ref, b_ref, keys_ref, s_ref, allp_ref):
    dims = (((1,), (0,)), ((), ()))
    w = w_ref[...]
    bias = b_ref[...]
    for c in range(NC):
        x_hi = x_ref[pl.ds(c * CH, CH), :].astype(jnp.bfloat16)
        logits = jax.lax.dot_general(x_hi, w, dims,
                                     preferred_element_type=jnp.float32)
        logits += bias
        _chunk_body(logits, keys_ref, s_ref, allp_ref, c)


def _sc_topk_body(keys_hbm, s_hbm, topp_hbm, topi_hbm,
                  keys_v, s_v, outp_v, outi_v):
    ncores = 2
    wid = lax.axis_index("s") * ncores + lax.axis_index("c")
    base = wid * TPW

    pltpu.sync_copy(keys_hbm.at[pl.ds(base * E, TPW * E)], keys_v)
    pltpu.sync_copy(s_hbm.at[pl.ds(base, TPW)], s_v)

    lane = lax.iota(jnp.int32, 16)
    lo8 = lane < 8

    def vsort(x):
        return plsc.sort_key_val(x, x)[0]

    def top16(a, b):
        return vsort(jnp.maximum(a, jnp.flip(b)))

    def token_top8(t):
        s0 = vsort(keys_v[pl.ds(t * E, 16)])
        s1 = vsort(keys_v[pl.ds(t * E + 16, 16)])
        s2 = vsort(keys_v[pl.ds(t * E + 32, 16)])
        s3 = vsort(keys_v[pl.ds(t * E + 48, 16)])
        return top16(top16(s0, s1), top16(s2, s3))

    def group(g, carry):
        sv = s_v[pl.ds(g * 16, 16)]
        for j in range(8):
            t0 = g * 16 + 2 * j
            ta = token_top8(t0)
            tb = token_top8(t0 + 1)
            ia = jnp.where(lo8, 15 - lane, 0)
            ib = jnp.where(lo8, 0, 23 - lane)
            kp = jnp.where(lo8,
                           ta.at[ia].get(mode="promise_in_bounds"),
                           tb.at[ib].get(mode="promise_in_bounds"))
            ti = 63 - (kp & 63)
            qf = (kp >> 6).astype(jnp.float32) * (2.0 ** -22)
            v = jnp.exp(qf)
            sum0 = jnp.sum(jnp.where(lo8, v, 0.0))
            sum1 = jnp.sum(jnp.where(lo8, 0.0, v))
            sums = jnp.where(lo8, sum0, sum1)
            si = jnp.where(lo8, 2 * j, 2 * j + 1)
            sp = sv.at[si].get(mode="promise_in_bounds")
            topp = v / (sums + 1e-8 * sp)
            outp_v[pl.ds(t0 * TOP_K, 16)] = topp
            outi_v[pl.ds(t0 * TOP_K, 16)] = ti
        return carry

    lax.fori_loop(0, TPW // 16, group, 0)

    pltpu.sync_copy(outp_v, topp_hbm.at[pl.ds(base * TOP_K, TPW * TOP_K)])
    pltpu.sync_copy(outi_v, topi_hbm.at[pl.ds(base * TOP_K, TPW * TOP_K)])


_sc_topk = functools.partial(
    pl.kernel,
    mesh=plsc.VectorSubcoreMesh(core_axis_name="c", subcore_axis_name="s"),
    compiler_params=pltpu.CompilerParams(needs_layout_passes=False),
    out_type=[
        jax.ShapeDtypeStruct((B * TOP_K,), jnp.float32),
        jax.ShapeDtypeStruct((B * TOP_K,), jnp.int32),
    ],
    scratch_types=[
        pltpu.VMEM((TPW * E,), jnp.int32),
        pltpu.VMEM((TPW,), jnp.float32),
        pltpu.VMEM((TPW * TOP_K,), jnp.float32),
        pltpu.VMEM((TPW * TOP_K,), jnp.int32),
    ],
)(_sc_topk_body)


@jax.jit
def kernel(x, W, b):
    w_hi = W.T.astype(jnp.bfloat16)
    b2 = b.reshape(1, E).astype(jnp.float32)

    grid = (B // BT,)
    out_shape = (
        jax.ShapeDtypeStruct((B, E), jnp.int32),
        jax.ShapeDtypeStruct((B, 1), jnp.float32),
        jax.ShapeDtypeStruct((B, E), jnp.float32),
    )
    keys, svec, allp = pl.pallas_call(
        _router_block,
        grid=grid,
        in_specs=[
            pl.BlockSpec((BT, D), lambda i: (i, 0)),
            pl.BlockSpec((D, E), lambda i: (0, 0)),
            pl.BlockSpec((1, E), lambda i: (0, 0)),
        ],
        out_specs=(
            pl.BlockSpec((BT, E), lambda i: (i, 0)),
            pl.BlockSpec((BT, 1), lambda i: (i, 0)),
            pl.BlockSpec((BT, E), lambda i: (i, 0)),
        ),
        out_shape=out_shape,
        compiler_params=pltpu.CompilerParams(
            dimension_semantics=("parallel",),
        ),
    )(x, w_hi, b2)

    topp_flat, topi_flat = _sc_topk(keys.reshape(-1), svec.reshape(-1))
    return (topp_flat.reshape(B, TOP_K), topi_flat.reshape(B, TOP_K), allp)


# speedup vs baseline: 1.4675x; 1.4675x over previous
import jax
import jax.numpy as jnp
from jax.experimental import pallas as pl
from jax.experimental.pallas import tpu as pltpu

B, D, E = 32768, 4096, 64
TOP_K = 8
BT = 1024
NC = 4
CH = BT // NC


def _chunk_epilogue(logits, topp_ref, topi_ref, allp_ref, c):
    m = jnp.max(logits, axis=1, keepdims=True)
    q = logits - m
    e = jnp.exp(q)
    s = jnp.sum(e, axis=1, keepdims=True)
    allp_ref[pl.ds(c * CH, CH), :] = e / s

    qt = q.T
    iota = jax.lax.broadcasted_iota(jnp.int32, (E, CH), 0)
    kq = (jnp.maximum(qt, -4.0) * (2.0 ** 23)).astype(jnp.int32)
    key = kq * 64 + (63 - iota)
    int_min = jnp.int32(-(2 ** 31))

    mks = []
    for _ in range(TOP_K):
        mk = jnp.max(key, axis=0, keepdims=True)
        key = jnp.where(key == mk, int_min, key)
        mks.append(mk)

    mkt = jnp.concatenate(mks, axis=0)
    tit = 63 - (mkt & 63)
    qf = (mkt >> 6).astype(jnp.float32) * (2.0 ** -23)
    tvt = jnp.exp(qf)

    tv = tvt.T / s
    norm = jnp.sum(tv, axis=1, keepdims=True) + 1e-8
    topp_ref[pl.ds(c * CH, CH), :] = tv / norm
    topi_ref[pl.ds(c * CH, CH), :] = tit.T


def _router_block(x_ref, w_ref, b_ref, topp_ref, topi_ref, allp_ref):
    dims = (((1,), (0,)), ((), ()))
    w = w_ref[...]
    bias = b_ref[...]
    for c in range(NC):
        x_hi = x_ref[pl.ds(c * CH, CH), :].astype(jnp.bfloat16)
        logits = jax.lax.dot_general(x_hi, w, dims,
                                     preferred_element_type=jnp.float32)
        logits += bias
        _chunk_epilogue(logits, topp_ref, topi_ref, allp_ref, c)


@jax.jit
def kernel(x, W, b):
    w_hi = W.T.astype(jnp.bfloat16)
    b2 = b.reshape(1, E).astype(jnp.float32)

    grid = (B // BT,)
    out_shape = (
        jax.ShapeDtypeStruct((B, TOP_K), jnp.float32),
        jax.ShapeDtypeStruct((B, TOP_K), jnp.int32),
        jax.ShapeDtypeStruct((B, E), jnp.float32),
    )
    topp, topi, allp = pl.pallas_call(
        _router_block,
        grid=grid,
        in_specs=[
            pl.BlockSpec((BT, D), lambda i: (i, 0)),
            pl.BlockSpec((D, E), lambda i: (0, 0)),
            pl.BlockSpec((1, E), lambda i: (0, 0)),
        ],
        out_specs=(
            pl.BlockSpec((BT, TOP_K), lambda i: (i, 0)),
            pl.BlockSpec((BT, TOP_K), lambda i: (i, 0)),
            pl.BlockSpec((BT, E), lambda i: (i, 0)),
        ),
        out_shape=out_shape,
        compiler_params=pltpu.CompilerParams(
            dimension_semantics=("parallel",),
        ),
    )(x, w_hi, b2)
    return topp, topi, allp
